# baseline (speedup 1.0000x reference)
import functools

import jax
import jax.numpy as jnp
from jax import lax
from jax.experimental import pallas as pl
from jax.experimental.pallas import tpu as pltpu
from jax.experimental.pallas import tpu_sc as plsc

_B = 16384
_L = 50
_D = 32
_NC = 2
_NS = 16
_NW = _NC * _NS
_BPW = _B // _NW
_STRIDE = 128
_GL = 56
_C = 2
_CR = _C * _GL
_NCHUNK = _BPW // _C
_IPW = _BPW * _STRIDE

_mesh = plsc.VectorSubcoreMesh(core_axis_name="c", subcore_axis_name="s")


@functools.partial(
    pl.kernel,
    mesh=_mesh,
    compiler_params=pltpu.CompilerParams(use_tc_tiling_on_sc=False),
    out_type=jax.ShapeDtypeStruct((_B, _D), jnp.float32),
    scratch_types=[
        pltpu.VMEM((_IPW,), jnp.int32),
        pltpu.VMEM((_CR, _D), jnp.float32),
        pltpu.VMEM((_CR, _D), jnp.float32),
        pltpu.VMEM((_BPW, _D), jnp.float32),
        pltpu.SemaphoreType.DMA,
        pltpu.SemaphoreType.DMA,
    ],
)
def _embed_mean(idx_hbm, table_hbm, out_hbm,
                idx_v, rows0, rows1, out_v, sem0, sem1):
    wid = lax.axis_index("s") * _NC + lax.axis_index("c")

    pltpu.sync_copy(idx_hbm.at[pl.ds(wid * _IPW, _IPW)], idx_v)

    row_b = (rows0, rows1)
    sem_b = (sem0, sem1)

    def chunk_copies(g, buf):
        copies = []
        for i in range(_C):
            copies.append(pltpu.make_async_copy(
                table_hbm.at[idx_v.at[pl.ds((g * _C + i) * _STRIDE, _GL)]],
                row_b[buf].at[pl.ds(i * _GL, _GL)], sem_b[buf]))
        return copies

    def load_chunk(g, buf):
        for c in chunk_copies(g, buf):
            c.start()

    def wait_chunk(g, buf):
        for c in chunk_copies(g, buf):
            c.wait()

    inv = jnp.float32(1.0 / _L)

    def reduce_chunk(g, buf):
        rows = row_b[buf]
        for i in range(_C):
            r0 = i * _GL
            a0 = jnp.zeros((16,), jnp.float32)
            a1 = jnp.zeros((16,), jnp.float32)
            b0 = jnp.zeros((16,), jnp.float32)
            b1 = jnp.zeros((16,), jnp.float32)
            for j in range(_L // 2):
                ra = r0 + 2 * j
                rb = ra + 1
                a0 = a0 + rows[ra, pl.ds(0, 16)]
                a1 = a1 + rows[ra, pl.ds(16, 16)]
                b0 = b0 + rows[rb, pl.ds(0, 16)]
                b1 = b1 + rows[rb, pl.ds(16, 16)]
            bag = g * _C + i
            out_v[bag, pl.ds(0, 16)] = (a0 + b0) * inv
            out_v[bag, pl.ds(16, 16)] = (a1 + b1) * inv

    load_chunk(0, 0)

    def pair(p, carry):
        for half in range(2):
            g = 2 * p + half
            cur = half
            nxt = 1 - half

            @pl.when(g + 1 < _NCHUNK)
            def _():
                load_chunk(g + 1, nxt)

            wait_chunk(g, cur)
            reduce_chunk(g, cur)
        return carry

    lax.fori_loop(0, _NCHUNK // 2, pair, 0)

    pltpu.sync_copy(out_v, out_hbm.at[pl.ds(wid * _BPW, _BPW)])


def kernel(ngrams, weight):
    ng = jnp.pad(ngrams.astype(jnp.int32), ((0, 0), (0, _STRIDE - _L)))
    return _embed_mean(ng.reshape(-1), weight)


# speedup vs baseline: 2.5313x; 2.5313x over previous
import functools

import jax
import jax.numpy as jnp
from jax import lax
from jax.experimental import pallas as pl
from jax.experimental.pallas import tpu as pltpu
from jax.experimental.pallas import tpu_sc as plsc

_B = 16384
_L = 50
_D = 32
_NC = 2
_NS = 16
_NW = _NC * _NS
_BPW = _B // _NW
_C = 4
_CR = _C * _L
_S0 = 96
_S1 = _CR - _S0
_NCHUNK = _BPW // _C
_IPW = _BPW * _L

_mesh = plsc.VectorSubcoreMesh(core_axis_name="c", subcore_axis_name="s")


@functools.partial(
    pl.kernel,
    mesh=_mesh,
    compiler_params=pltpu.CompilerParams(
        needs_layout_passes=False, use_tc_tiling_on_sc=True),
    out_type=jax.ShapeDtypeStruct((_B * _L,), jnp.int32),
    scratch_types=[
        pltpu.VMEM((_BPW, _L), jnp.int32),
        pltpu.VMEM((_IPW,), jnp.int32),
        pltpu.SemaphoreType.DMA,
    ],
)
def _flatten_sc(ng_hbm, out_hbm, ng_v, flat_v, sem):
    wid = lax.axis_index("s") * _NC + lax.axis_index("c")
    r0 = wid * _BPW
    pltpu.async_copy(ng_hbm.at[pl.ds(r0, _BPW)], ng_v, sem).wait()
    iota = lax.iota(jnp.int32, 16)

    def row(r, carry):
        base = r * _L
        for off in (0, 16, 32, 34):
            v = ng_v[r, pl.ds(off, 16)]
            plsc.store_scatter(flat_v, [base + off + iota], v)
        return carry

    lax.fori_loop(0, _BPW, row, 0)
    pltpu.async_copy(flat_v, out_hbm.at[pl.ds(wid * _IPW, _IPW)], sem).wait()


@functools.partial(
    pl.kernel,
    mesh=_mesh,
    compiler_params=pltpu.CompilerParams(
        needs_layout_passes=False, use_tc_tiling_on_sc=False),
    out_type=jax.ShapeDtypeStruct((_B, _D), jnp.float32),
    scratch_types=[
        pltpu.VMEM((_IPW,), jnp.int32),
        pltpu.VMEM((_CR, _D), jnp.float32),
        pltpu.VMEM((_CR, _D), jnp.float32),
        pltpu.VMEM((_BPW, _D), jnp.float32),
        pltpu.SemaphoreType.DMA,
        pltpu.SemaphoreType.DMA,
    ],
)
def _embed_mean(idx_hbm, table_hbm, out_hbm,
                idx_v, rows0, rows1, out_v, sem0, sem1):
    wid = lax.axis_index("s") * _NC + lax.axis_index("c")

    pltpu.sync_copy(idx_hbm.at[pl.ds(wid * _IPW, _IPW)], idx_v)

    row_b = (rows0, rows1)
    sem_b = (sem0, sem1)

    def load_chunk(g, buf):
        base = g * _CR
        pltpu.async_copy(
            table_hbm.at[idx_v.at[pl.ds(base, _S0)]],
            row_b[buf].at[pl.ds(0, _S0)], sem_b[buf])
        pltpu.async_copy(
            table_hbm.at[idx_v.at[pl.ds(base + _S0, _S1)]],
            row_b[buf].at[pl.ds(_S0, _S1)], sem_b[buf])

    def wait_chunk(g, buf):
        base = g * _CR
        pltpu.make_async_copy(
            table_hbm.at[idx_v.at[pl.ds(base, _S0)]],
            row_b[buf].at[pl.ds(0, _S0)], sem_b[buf]).wait()
        pltpu.make_async_copy(
            table_hbm.at[idx_v.at[pl.ds(base + _S0, _S1)]],
            row_b[buf].at[pl.ds(_S0, _S1)], sem_b[buf]).wait()

    inv = jnp.float32(1.0 / _L)

    def reduce_chunk(g, buf):
        rows = row_b[buf]
        for i in range(_C):
            r0 = i * _L
            a0 = jnp.zeros((16,), jnp.float32)
            a1 = jnp.zeros((16,), jnp.float32)
            b0 = jnp.zeros((16,), jnp.float32)
            b1 = jnp.zeros((16,), jnp.float32)
            for j in range(_L // 2):
                ra = r0 + 2 * j
                rb = ra + 1
                a0 = a0 + rows[ra, pl.ds(0, 16)]
                a1 = a1 + rows[ra, pl.ds(16, 16)]
                b0 = b0 + rows[rb, pl.ds(0, 16)]
                b1 = b1 + rows[rb, pl.ds(16, 16)]
            bag = g * _C + i
            out_v[bag, pl.ds(0, 16)] = (a0 + b0) * inv
            out_v[bag, pl.ds(16, 16)] = (a1 + b1) * inv

    load_chunk(0, 0)

    def pair(p, carry):
        for half in range(2):
            g = 2 * p + half
            cur = half
            nxt = 1 - half

            @pl.when(g + 1 < _NCHUNK)
            def _():
                load_chunk(g + 1, nxt)

            wait_chunk(g, cur)
            reduce_chunk(g, cur)
        return carry

    lax.fori_loop(0, _NCHUNK // 2, pair, 0)

    pltpu.sync_copy(out_v, out_hbm.at[pl.ds(wid * _BPW, _BPW)])


def kernel(ngrams, weight):
    ng = _flatten_sc(ngrams.astype(jnp.int32))
    return _embed_mean(ng, weight)


# speedup vs baseline: 2.5517x; 1.0080x over previous
import functools

import jax
import jax.numpy as jnp
from jax import lax
from jax.experimental import pallas as pl
from jax.experimental.pallas import tpu as pltpu
from jax.experimental.pallas import tpu_sc as plsc

_B = 16384
_L = 50
_D = 32
_NC = 2
_NS = 16
_NW = _NC * _NS
_BPW = _B // _NW
_C = 4
_CR = _C * _L
_S0 = 96
_S1 = _CR - _S0
_NCHUNK = _BPW // _C
_IPW = _BPW * _L

_mesh = plsc.VectorSubcoreMesh(core_axis_name="c", subcore_axis_name="s")


@functools.partial(
    pl.kernel,
    mesh=_mesh,
    compiler_params=pltpu.CompilerParams(
        needs_layout_passes=False, use_tc_tiling_on_sc=True),
    out_type=jax.ShapeDtypeStruct((_B * _L,), jnp.int32),
    scratch_types=[
        pltpu.VMEM((_BPW, _L), jnp.int32),
        pltpu.VMEM((_IPW,), jnp.int32),
        pltpu.SemaphoreType.DMA,
    ],
)
def _flatten_sc(ng_hbm, out_hbm, ng_v, flat_v, sem):
    wid = lax.axis_index("s") * _NC + lax.axis_index("c")
    r0 = wid * _BPW
    pltpu.async_copy(ng_hbm.at[pl.ds(r0, _BPW)], ng_v, sem).wait()
    iota = lax.iota(jnp.int32, 16)

    def row(r, carry):
        base = r * _L
        for off in (0, 16, 32, 34):
            v = ng_v[r, pl.ds(off, 16)]
            plsc.store_scatter(flat_v, [base + off + iota], v)
        return carry

    lax.fori_loop(0, _BPW, row, 0)
    pltpu.async_copy(flat_v, out_hbm.at[pl.ds(wid * _IPW, _IPW)], sem).wait()


@functools.partial(
    pl.kernel,
    mesh=_mesh,
    compiler_params=pltpu.CompilerParams(
        needs_layout_passes=False, use_tc_tiling_on_sc=False),
    out_type=jax.ShapeDtypeStruct((_B, 128), jnp.float32),
    scratch_types=[
        pltpu.VMEM((_IPW,), jnp.int32),
        pltpu.VMEM((_CR, _D), jnp.float32),
        pltpu.VMEM((_CR, _D), jnp.float32),
        pltpu.VMEM((_BPW, 128), jnp.float32),
        pltpu.SemaphoreType.DMA,
        pltpu.SemaphoreType.DMA,
    ],
)
def _embed_mean(idx_hbm, table_hbm, out_hbm,
                idx_v, rows0, rows1, out_v, sem0, sem1):
    wid = lax.axis_index("s") * _NC + lax.axis_index("c")

    pltpu.sync_copy(idx_hbm.at[pl.ds(wid * _IPW, _IPW)], idx_v)

    row_b = (rows0, rows1)
    sem_b = (sem0, sem1)

    def load_chunk(g, buf):
        base = g * _CR
        pltpu.async_copy(
            table_hbm.at[idx_v.at[pl.ds(base, _S0)]],
            row_b[buf].at[pl.ds(0, _S0)], sem_b[buf])
        pltpu.async_copy(
            table_hbm.at[idx_v.at[pl.ds(base + _S0, _S1)]],
            row_b[buf].at[pl.ds(_S0, _S1)], sem_b[buf])

    def wait_chunk(g, buf):
        base = g * _CR
        pltpu.make_async_copy(
            table_hbm.at[idx_v.at[pl.ds(base, _S0)]],
            row_b[buf].at[pl.ds(0, _S0)], sem_b[buf]).wait()
        pltpu.make_async_copy(
            table_hbm.at[idx_v.at[pl.ds(base + _S0, _S1)]],
            row_b[buf].at[pl.ds(_S0, _S1)], sem_b[buf]).wait()

    inv = jnp.float32(1.0 / _L)

    def reduce_chunk(g, buf):
        rows = row_b[buf]
        for i in range(_C):
            r0 = i * _L
            a0 = jnp.zeros((16,), jnp.float32)
            a1 = jnp.zeros((16,), jnp.float32)
            b0 = jnp.zeros((16,), jnp.float32)
            b1 = jnp.zeros((16,), jnp.float32)
            for j in range(_L // 2):
                ra = r0 + 2 * j
                rb = ra + 1
                a0 = a0 + rows[ra, pl.ds(0, 16)]
                a1 = a1 + rows[ra, pl.ds(16, 16)]
                b0 = b0 + rows[rb, pl.ds(0, 16)]
                b1 = b1 + rows[rb, pl.ds(16, 16)]
            bag = g * _C + i
            out_v[bag, pl.ds(0, 16)] = (a0 + b0) * inv
            out_v[bag, pl.ds(16, 16)] = (a1 + b1) * inv

    load_chunk(0, 0)

    def pair(p, carry):
        for half in range(2):
            g = 2 * p + half
            cur = half
            nxt = 1 - half

            @pl.when(g + 1 < _NCHUNK)
            def _():
                load_chunk(g + 1, nxt)

            wait_chunk(g, cur)
            reduce_chunk(g, cur)
        return carry

    lax.fori_loop(0, _NCHUNK // 2, pair, 0)

    pltpu.sync_copy(out_v, out_hbm.at[pl.ds(wid * _BPW, _BPW)])


def kernel(ngrams, weight):
    ng = _flatten_sc(ngrams.astype(jnp.int32))
    return _embed_mean(ng, weight)[:, :_D]
